# baseline (speedup 1.0000x reference)
import functools

import jax
import jax.numpy as jnp
from jax import lax
from jax.experimental import pallas as pl
from jax.experimental.pallas import tpu as pltpu
from jax.experimental.pallas import tpu_sc as plsc

N = 320000
C_IN = 128
UNITS = 64
NKEYS = 16 ** 4

NC, NS, L = 2, 16, 16
NW = NC * NS

MM_BLK = 2560

SHARD = 1024
SHARDS = NKEYS // (NW * SHARD)
KCH = 8000
CAP = 512
HI = CAP - L
GB = 400


def _mm_body(x_ref, bxyz_ref, w_ref, b_ref, o_ref, k_ref):
    o_ref[...] = (
        jnp.dot(x_ref[...], w_ref[...], preferred_element_type=jnp.float32)
        + b_ref[...]
    )
    c = bxyz_ref[...]
    k = c[:, 0] * 4096 + c[:, 1] * 256 + c[:, 2] * 16 + c[:, 3]
    k_ref[...] = k.reshape(-1, 1)


def _linear_and_keys(inputs, bxyz_indx, W, b):
    return pl.pallas_call(
        _mm_body,
        grid=(N // MM_BLK,),
        in_specs=[
            pl.BlockSpec((MM_BLK, C_IN), lambda i: (i, 0)),
            pl.BlockSpec((MM_BLK, 4), lambda i: (i, 0)),
            pl.BlockSpec((C_IN, UNITS), lambda i: (0, 0)),
            pl.BlockSpec((1, UNITS), lambda i: (0, 0)),
        ],
        out_specs=[
            pl.BlockSpec((MM_BLK, UNITS), lambda i: (i, 0)),
            pl.BlockSpec((MM_BLK, 1), lambda i: (i, 0)),
        ],
        out_shape=[
            jax.ShapeDtypeStruct((N, UNITS), jnp.float32),
            jax.ShapeDtypeStruct((N, 1), jnp.int32),
        ],
    )(inputs, bxyz_indx, W, b.reshape(1, UNITS))


_SC_MESH = plsc.VectorSubcoreMesh(
    core_axis_name="c", subcore_axis_name="s", num_cores=NC, num_subcores=NS
)

_SC_PARAMS = pltpu.CompilerParams(
    needs_layout_passes=False, use_tc_tiling_on_sc=False
)


@functools.partial(
    pl.kernel,
    out_type=jax.ShapeDtypeStruct((NKEYS, UNITS), jnp.float32),
    mesh=_SC_MESH,
    compiler_params=_SC_PARAMS,
    scratch_types=[
        pltpu.VMEM((KCH,), jnp.int32),
        pltpu.VMEM((CAP,), jnp.int32),
        pltpu.VMEM((CAP + L,), jnp.int32),
        pltpu.VMEM((CAP, UNITS), jnp.float32),
        pltpu.VMEM((SHARD, UNITS), jnp.float32),
        pltpu.SemaphoreType.DMA,
    ],
)
def _scatter_max(keys_hbm, x_hbm, table_hbm, kbuf, pidx, pkey, rows, table_v,
                 sem):
    wid = lax.axis_index("s") * NC + lax.axis_index("c")
    neg = jnp.full((L,), -jnp.inf, dtype=jnp.float32)

    def pinit(i, carry):
        pidx[pl.ds(i * L, L)] = jnp.zeros((L,), jnp.int32)
        return carry

    lax.fori_loop(0, CAP // L, pinit, 0)

    for s in range(SHARDS):
        key_base = (wid * SHARDS + s) * SHARD

        def tinit(i, carry):
            for f in range(UNITS // L):
                table_v[i, pl.ds(f * L, L)] = neg
            return carry

        lax.fori_loop(0, SHARD, tinit, 0)

        def flush(cnt):
            pltpu.async_copy(x_hbm.at[pidx], rows, sem).wait()

            def upd(i, carry):
                lk = pkey[pl.ds(i, L)][0]
                for f in range(UNITS // L):
                    sl = pl.ds(f * L, L)
                    table_v[lk, sl] = jnp.maximum(table_v[lk, sl],
                                                  rows[i, sl])
                return carry

            lax.fori_loop(0, cnt, upd, 0)

        def chunk_body(c, cursor):
            pltpu.sync_copy(keys_hbm.at[pl.ds(c * KCH, KCH)], kbuf)

            def vec_body(v, cur):
                def do_flush(cc):
                    flush(cc)
                    return 0

                kv = kbuf[pl.ds(v * L, L)]
                lkv = kv - key_base
                m = (lkv >= 0) & (lkv < SHARD)

                def append(cc):
                    cc = lax.cond(cc >= HI, do_flush, lambda a: a, cc)
                    cnt = jnp.sum(m.astype(jnp.int32))
                    gidx = c * KCH + v * L + lax.iota(jnp.int32, L)
                    plsc.store_compressed(pidx.at[pl.ds(cc, L)], gidx,
                                          mask=m)
                    plsc.store_compressed(pkey.at[pl.ds(cc, L)], lkv,
                                          mask=m)
                    return cc + cnt

                return lax.cond(jnp.any(m), append, lambda a: a, cur)

            return lax.fori_loop(0, KCH // L, vec_body, cursor)

        cursor = lax.fori_loop(0, N // KCH, chunk_body, 0)
        flush(cursor)
        pltpu.sync_copy(table_v, table_hbm.at[pl.ds(key_base, SHARD)])


@functools.partial(
    pl.kernel,
    out_type=jax.ShapeDtypeStruct((N, UNITS), jnp.float32),
    mesh=_SC_MESH,
    compiler_params=_SC_PARAMS,
    scratch_types=[
        pltpu.VMEM((GB,), jnp.int32),
        pltpu.VMEM((GB, UNITS), jnp.float32),
        pltpu.SemaphoreType.DMA,
    ],
)
def _gather_back(keys_hbm, table_hbm, out_hbm, gkey, grow, sem):
    wid = lax.axis_index("s") * NC + lax.axis_index("c")
    base = wid * (N // NW)

    def chunk(g, carry):
        off = base + g * GB
        pltpu.sync_copy(keys_hbm.at[pl.ds(off, GB)], gkey)
        pltpu.async_copy(table_hbm.at[gkey], grow, sem).wait()
        pltpu.sync_copy(grow, out_hbm.at[pl.ds(off, GB)])
        return carry

    lax.fori_loop(0, (N // NW) // GB, chunk, 0)


def kernel(inputs, bxyz_indx, W, b):
    x, keys2d = _linear_and_keys(inputs, bxyz_indx, W, b)
    keys = keys2d.reshape(-1)
    table = _scatter_max(keys, x)
    g = _gather_back(keys, table)
    return jnp.concatenate([x, g], axis=1)


# speedup vs baseline: 1.1418x; 1.1418x over previous
import functools

import jax
import jax.numpy as jnp
from jax import lax
from jax.experimental import pallas as pl
from jax.experimental.pallas import tpu as pltpu
from jax.experimental.pallas import tpu_sc as plsc

N = 320000
C_IN = 128
UNITS = 64
NKEYS = 16 ** 4

NC, NS, L = 2, 16, 16
NW = NC * NS

MM_BLK = 2560

SHARD = 1024
SHARDS = NKEYS // (NW * SHARD)
KCH = 8000
CAP = 512
HI = CAP - L
GB = 400


def _mm_body(x_ref, bxyz_ref, w_ref, b_ref, o_ref, k_ref):
    o_ref[...] = (
        jnp.dot(x_ref[...], w_ref[...], preferred_element_type=jnp.float32)
        + b_ref[...]
    )
    c = bxyz_ref[...]
    k = c[:, 0] * 4096 + c[:, 1] * 256 + c[:, 2] * 16 + c[:, 3]
    k_ref[...] = k.reshape(-1, 1)


def _linear_and_keys(inputs, bxyz_indx, W, b):
    return pl.pallas_call(
        _mm_body,
        grid=(N // MM_BLK,),
        in_specs=[
            pl.BlockSpec((MM_BLK, C_IN), lambda i: (i, 0)),
            pl.BlockSpec((MM_BLK, 4), lambda i: (i, 0)),
            pl.BlockSpec((C_IN, UNITS), lambda i: (0, 0)),
            pl.BlockSpec((1, UNITS), lambda i: (0, 0)),
        ],
        out_specs=[
            pl.BlockSpec((MM_BLK, UNITS), lambda i: (i, 0)),
            pl.BlockSpec((MM_BLK, 1), lambda i: (i, 0)),
        ],
        out_shape=[
            jax.ShapeDtypeStruct((N, UNITS), jnp.float32),
            jax.ShapeDtypeStruct((N, 1), jnp.int32),
        ],
    )(inputs, bxyz_indx, W, b.reshape(1, UNITS))


_SC_MESH = plsc.VectorSubcoreMesh(
    core_axis_name="c", subcore_axis_name="s", num_cores=NC, num_subcores=NS
)

_SC_PARAMS = pltpu.CompilerParams(
    needs_layout_passes=False, use_tc_tiling_on_sc=False
)


@functools.partial(
    pl.kernel,
    out_type=jax.ShapeDtypeStruct((NKEYS, UNITS), jnp.float32),
    mesh=_SC_MESH,
    compiler_params=_SC_PARAMS,
    scratch_types=[
        pltpu.VMEM((KCH,), jnp.int32),
        pltpu.VMEM((CAP,), jnp.int32),
        pltpu.VMEM((CAP + L,), jnp.int32),
        pltpu.VMEM((CAP, UNITS), jnp.float32),
        pltpu.VMEM((SHARD, UNITS), jnp.float32),
        pltpu.SemaphoreType.DMA,
    ],
)
def _scatter_max(keys_hbm, x_hbm, table_hbm, kbuf, pidx, pkey, rows, table_v,
                 sem):
    wid = lax.axis_index("s") * NC + lax.axis_index("c")
    neg = jnp.full((L,), -jnp.inf, dtype=jnp.float32)

    def pinit(i, carry):
        pidx[pl.ds(i * L, L)] = jnp.zeros((L,), jnp.int32)
        return carry

    lax.fori_loop(0, CAP // L, pinit, 0)

    for s in range(SHARDS):
        key_base = (wid * SHARDS + s) * SHARD

        def tinit(i, carry):
            for f in range(UNITS // L):
                table_v[i, pl.ds(f * L, L)] = neg
            return carry

        lax.fori_loop(0, SHARD, tinit, 0)

        def flush(cnt):
            pltpu.async_copy(x_hbm.at[pidx], rows, sem).wait()

            def upd(i, carry):
                lk = pkey[pl.ds(i, L)][0]
                for f in range(UNITS // L):
                    sl = pl.ds(f * L, L)
                    table_v[lk, sl] = jnp.maximum(table_v[lk, sl],
                                                  rows[i, sl])
                return carry

            lax.fori_loop(0, cnt, upd, 0)

        def chunk_body(c, cursor):
            pltpu.sync_copy(keys_hbm.at[pl.ds(c * KCH, KCH)], kbuf)

            def vec_body(v, cur):
                def do_flush(cc):
                    flush(cc)
                    return 0

                cur = lax.cond(cur >= HI, do_flush, lambda cc: cc, cur)
                kv = kbuf[pl.ds(v * L, L)]
                lkv = kv - key_base
                m = (lkv >= 0) & (lkv < SHARD)
                cnt = jnp.sum(m.astype(jnp.int32))
                gidx = c * KCH + v * L + lax.iota(jnp.int32, L)
                plsc.store_compressed(pidx.at[pl.ds(cur, L)], gidx, mask=m)
                plsc.store_compressed(pkey.at[pl.ds(cur, L)], lkv, mask=m)
                return cur + cnt

            return lax.fori_loop(0, KCH // L, vec_body, cursor)

        cursor = lax.fori_loop(0, N // KCH, chunk_body, 0)
        flush(cursor)
        pltpu.sync_copy(table_v, table_hbm.at[pl.ds(key_base, SHARD)])


@functools.partial(
    pl.kernel,
    out_type=jax.ShapeDtypeStruct((N, UNITS), jnp.float32),
    mesh=_SC_MESH,
    compiler_params=_SC_PARAMS,
    scratch_types=[
        pltpu.VMEM((GB,), jnp.int32),
        pltpu.VMEM((GB, UNITS), jnp.float32),
        pltpu.SemaphoreType.DMA,
    ],
)
def _gather_back(keys_hbm, table_hbm, out_hbm, gkey, grow, sem):
    wid = lax.axis_index("s") * NC + lax.axis_index("c")
    base = wid * (N // NW)

    def chunk(g, carry):
        off = base + g * GB
        pltpu.sync_copy(keys_hbm.at[pl.ds(off, GB)], gkey)
        pltpu.async_copy(table_hbm.at[gkey], grow, sem).wait()
        pltpu.sync_copy(grow, out_hbm.at[pl.ds(off, GB)])
        return carry

    lax.fori_loop(0, (N // NW) // GB, chunk, 0)


def kernel(inputs, bxyz_indx, W, b):
    x, keys2d = _linear_and_keys(inputs, bxyz_indx, W, b)
    keys = keys2d.reshape(-1)
    table = _scatter_max(keys, x)
    g = _gather_back(keys, table)
    return jnp.concatenate([x, g], axis=1)
